# Initial kernel scaffold; baseline (speedup 1.0000x reference)
import jax
import jax.numpy as jnp
from jax.experimental import pallas as pl


def kernel(x_enc, x_mark_enc, x_dec, x_mark_dec, conv_w, temp_w, qk_w, v_w, out_w, out_b, n1_s, n1_b, ff1_w, ff1_b, ff2_w, ff2_b, n2_s, n2_b, norm_s, norm_b, proj_w, proj_b):
    raise NotImplementedError("write your pallas kernel here")


# speedup vs baseline: 1.2266x; 1.2266x over previous
import functools
import numpy as np

import jax
import jax.numpy as jnp
from jax import lax
from jax.experimental import pallas as pl
from jax.experimental.pallas import tpu as pltpu

B = 1
SEQ = 2048
PRED = 1024
ENC_IN = 7
D = 768
H = 12
DH = 64
DFF = 3072
NL = 2
BS = 64
NH = 4
T = SEQ + PRED
NC = T // BS
NPAIR = H * NH

_INTERPRET = False


def _pc(body, out_shape, in_specs, out_specs, grid=None, **kw):
    if grid is not None:
        kw["grid"] = grid
    return pl.pallas_call(
        body,
        out_shape=out_shape,
        in_specs=in_specs,
        out_specs=out_specs,
        interpret=_INTERPRET,
        **kw,
    )


def _positional_encoding(t, d):
    pos = np.arange(t, dtype=np.float32)[:, None]
    div = np.exp(np.arange(0, d, 2, dtype=np.float32) * -(np.log(10000.0) / d))
    pe = np.zeros((t, d), dtype=np.float32)
    pe[:, 0::2] = np.sin(pos * div)
    pe[:, 1::2] = np.cos(pos * div)
    return jnp.asarray(pe)


def _embed_body(xcat_ref, w_ref, pe_ref, out_ref):
    out_ref[...] = (
        jnp.dot(xcat_ref[...], w_ref[...], preferred_element_type=jnp.float32)
        + pe_ref[...]
    )


def _embed(xcat, w, pe):
    return _pc(
        _embed_body,
        jax.ShapeDtypeStruct((T, D), jnp.float32),
        [pl.BlockSpec(xcat.shape, lambda: (0, 0)),
         pl.BlockSpec(w.shape, lambda: (0, 0)),
         pl.BlockSpec(pe.shape, lambda: (0, 0))],
        pl.BlockSpec((T, D), lambda: (0, 0)),
    )(xcat, w, pe)


def _qkvh_body(hdd_ref, w_ref, rot_ref, qkv_ref, bkt_ref):
    qkv = jnp.dot(hdd_ref[...], w_ref[0], preferred_element_type=jnp.float32)
    qkv_ref[0] = qkv
    qk = qkv[:, :DH]
    s = jnp.dot(qk, rot_ref[...], preferred_element_type=jnp.float32)
    rows = []
    for n in range(NH):
        sn = s[:, n * (NC // 2):(n + 1) * (NC // 2)]
        cat = jnp.concatenate([sn, -sn], axis=1)
        bn = jnp.argmax(cat, axis=1).astype(jnp.int32)
        rows.append(bn.reshape(1, T))
    bkt_ref[0] = jnp.concatenate(rows, axis=0)


def _qkv_hash(hdd, wqkv, rot):
    return _pc(
        _qkvh_body,
        (jax.ShapeDtypeStruct((H, T, 2 * DH), jnp.float32),
         jax.ShapeDtypeStruct((H, NH, T), jnp.int32)),
        [pl.BlockSpec((T, D), lambda h: (0, 0)),
         pl.BlockSpec((1, D, 2 * DH), lambda h: (h, 0, 0)),
         pl.BlockSpec((DH, NH * (NC // 2)), lambda h: (0, 0))],
        (pl.BlockSpec((1, T, 2 * DH), lambda h: (h, 0, 0)),
         pl.BlockSpec((1, NH, T), lambda h: (h, 0, 0))),
        grid=(H,),
    )(hdd, wqkv, rot)


def _sort_gather(buckets, qkv):
    ticker = jnp.arange(T, dtype=jnp.int32)
    key = buckets * T + ticker[None, None, :]
    sticker = jnp.argsort(key, axis=-1).astype(jnp.int32)
    pos = jnp.argsort(sticker, axis=-1).astype(jnp.int32)
    sorted_qkv = jnp.take_along_axis(
        qkv[:, None, :, :], sticker[..., None], axis=2)
    return (sorted_qkv.reshape(NPAIR, T, 2 * DH),
            sticker.reshape(NPAIR, 1, T),
            pos.reshape(NPAIR, 1, T))


def _ungather(so, sl, pos):
    p2 = pos.reshape(NPAIR, T)
    o = jnp.take_along_axis(so, p2[..., None], axis=1)
    lg = jnp.take_along_axis(sl.reshape(NPAIR, T), p2, axis=1)
    return o, lg.reshape(NPAIR, 1, T)


def _attn_body(qkv_ref, so_ref, sl_ref):
    qkv = qkv_ref[0]
    q = qkv[:, :DH]
    v = qkv[:, DH:]
    nrm = jnp.sqrt(jnp.sum(q * q, axis=1, keepdims=True)) + 1e-9
    k = q / nrm
    bq = q.reshape(NC, BS, DH)
    bk = k.reshape(NC, BS, DH)
    bv = v.reshape(NC, BS, DH)
    kp = jnp.concatenate([bk[NC - 1:], bk[:NC - 1]], axis=0)
    vp = jnp.concatenate([bv[NC - 1:], bv[:NC - 1]], axis=0)
    bk2 = jnp.concatenate([kp, bk], axis=1)
    bv2 = jnp.concatenate([vp, bv], axis=1)
    dots = lax.dot_general(
        bq, bk2, (((2,), (2,)), ((0,), (0,))),
        preferred_element_type=jnp.float32) / jnp.sqrt(jnp.float32(DH))
    ri = lax.broadcasted_iota(jnp.int32, (NC, BS, 2 * BS), 1)
    ci = lax.broadcasted_iota(jnp.int32, (NC, BS, 2 * BS), 2)
    dots = jnp.where(ci == ri + BS, -1e5, dots)
    m = jnp.max(dots, axis=-1, keepdims=True)
    lse = m + jnp.log(jnp.sum(jnp.exp(dots - m), axis=-1, keepdims=True))
    probs = jnp.exp(dots - lse)
    bo = lax.dot_general(
        probs, bv2, (((2,), (1,)), ((0,), (0,))),
        preferred_element_type=jnp.float32)
    so_ref[0] = bo.reshape(T, DH)
    sl_ref[0] = lse


def _attention(sorted_qkv):
    return _pc(
        _attn_body,
        (jax.ShapeDtypeStruct((NPAIR, T, DH), jnp.float32),
         jax.ShapeDtypeStruct((NPAIR, NC, BS, 1), jnp.float32)),
        [pl.BlockSpec((1, T, 2 * DH), lambda p: (p, 0, 0))],
        (pl.BlockSpec((1, T, DH), lambda p: (p, 0, 0)),
         pl.BlockSpec((1, NC, BS, 1), lambda p: (p, 0, 0, 0))),
        grid=(NPAIR,),
    )(sorted_qkv)


def _combine_body(o_ref, lg_ref, out_ref):
    cols = []
    for g in range(2):
        lg = lg_ref[g * NH:(g + 1) * NH, 0, :]
        m = jnp.max(lg, axis=0, keepdims=True)
        e = jnp.exp(lg - m)
        w = e / jnp.sum(e, axis=0, keepdims=True)
        o = o_ref[g * NH:(g + 1) * NH]
        cols.append(jnp.sum(o * w[:, :, None], axis=0))
    out_ref[...] = jnp.concatenate(cols, axis=1)


def _combine(o, lg):
    return _pc(
        _combine_body,
        jax.ShapeDtypeStruct((T, D), jnp.float32),
        [pl.BlockSpec((2 * NH, T, DH), lambda j: (j, 0, 0)),
         pl.BlockSpec((2 * NH, 1, T), lambda j: (j, 0, 0))],
        pl.BlockSpec((T, 2 * DH), lambda j: (0, j)),
        grid=(H // 2,),
    )(o, lg)


def _ln(x, s, b):
    m = jnp.mean(x, axis=-1, keepdims=True)
    v = jnp.mean((x - m) ** 2, axis=-1, keepdims=True)
    return (x - m) / jnp.sqrt(v + 1e-5) * s + b


def _outproj_body(ao_ref, w_ref, b_ref, hdd_ref, s_ref, bb_ref, out_ref):
    y = jnp.dot(ao_ref[...], w_ref[...], preferred_element_type=jnp.float32)
    y = y + b_ref[...] + hdd_ref[...]
    out_ref[...] = _ln(y, s_ref[...], bb_ref[...])


def _outproj_ln(ao, w, b, hdd, s, bb):
    return _pc(
        _outproj_body,
        jax.ShapeDtypeStruct((T, D), jnp.float32),
        [pl.BlockSpec((T, D), lambda: (0, 0)),
         pl.BlockSpec((D, D), lambda: (0, 0)),
         pl.BlockSpec((1, D), lambda: (0, 0)),
         pl.BlockSpec((T, D), lambda: (0, 0)),
         pl.BlockSpec((1, D), lambda: (0, 0)),
         pl.BlockSpec((1, D), lambda: (0, 0))],
        pl.BlockSpec((T, D), lambda: (0, 0)),
    )(ao, w, b, hdd, s, bb)


def _ffn1_body(x_ref, w_ref, b_ref, y_ref):
    h = jnp.dot(x_ref[...], w_ref[...], preferred_element_type=jnp.float32)
    h = h + b_ref[...]
    y_ref[...] = 0.5 * h * (1.0 + lax.erf(h / np.float32(np.sqrt(2.0))))


def _ffn1(x, w, b):
    nb = 4
    return _pc(
        _ffn1_body,
        jax.ShapeDtypeStruct((T, DFF), jnp.float32),
        [pl.BlockSpec((T, D), lambda j: (0, 0)),
         pl.BlockSpec((D, DFF // nb), lambda j: (0, j)),
         pl.BlockSpec((1, DFF // nb), lambda j: (0, j))],
        pl.BlockSpec((T, DFF // nb), lambda j: (0, j)),
        grid=(nb,),
    )(x, w, b)


def _ffn2_body(y_ref, w_ref, b_ref, x1_ref, s_ref, bb_ref, out_ref, *, nb):
    kidx = pl.program_id(0)
    part = jnp.dot(y_ref[...], w_ref[...], preferred_element_type=jnp.float32)

    @pl.when(kidx == 0)
    def _():
        out_ref[...] = part

    @pl.when(kidx > 0)
    def _():
        out_ref[...] += part

    @pl.when(kidx == nb - 1)
    def _():
        z = out_ref[...] + b_ref[...] + x1_ref[...]
        out_ref[...] = _ln(z, s_ref[...], bb_ref[...])


def _ffn2(y, w, b, x1, s, bb):
    nb = 8
    return _pc(
        functools.partial(_ffn2_body, nb=nb),
        jax.ShapeDtypeStruct((T, D), jnp.float32),
        [pl.BlockSpec((T, DFF // nb), lambda k: (0, k)),
         pl.BlockSpec((DFF // nb, D), lambda k: (k, 0)),
         pl.BlockSpec((1, D), lambda k: (0, 0)),
         pl.BlockSpec((T, D), lambda k: (0, 0)),
         pl.BlockSpec((1, D), lambda k: (0, 0)),
         pl.BlockSpec((1, D), lambda k: (0, 0))],
        pl.BlockSpec((T, D), lambda k: (0, 0)),
        grid=(nb,),
    )(y, w, b, x1, s, bb)


def _final_body(x_ref, s_ref, b_ref, w_ref, pb_ref, out_ref):
    x = _ln(x_ref[...], s_ref[...], b_ref[...])
    out_ref[...] = (jnp.dot(x, w_ref[...], preferred_element_type=jnp.float32)
                    + pb_ref[...])


def _final(x, s, b, w, pb, cout):
    return _pc(
        _final_body,
        jax.ShapeDtypeStruct((T, cout), jnp.float32),
        [pl.BlockSpec((T, D), lambda: (0, 0)),
         pl.BlockSpec((1, D), lambda: (0, 0)),
         pl.BlockSpec((1, D), lambda: (0, 0)),
         pl.BlockSpec((D, cout), lambda: (0, 0)),
         pl.BlockSpec((1, cout), lambda: (0, 0))],
        pl.BlockSpec((T, cout), lambda: (0, 0)),
    )(x, s, b, w, pb)


def kernel(x_enc, x_mark_enc, x_dec, x_mark_dec, conv_w, temp_w, qk_w, v_w,
           out_w, out_b, n1_s, n1_b, ff1_w, ff1_b, ff2_w, ff2_b, n2_s, n2_b,
           norm_s, norm_b, proj_w, proj_b):
    cout = proj_w.shape[0]
    x = jnp.concatenate([x_enc[0], x_dec[0, -PRED:, :]], axis=0)
    xm = jnp.concatenate([x_mark_enc[0], x_mark_dec[0, -PRED:, :]], axis=0)
    xprev = jnp.roll(x, 1, axis=0)
    xnext = jnp.roll(x, -1, axis=0)
    xcat = jnp.concatenate([xprev, x, xnext, xm], axis=1)
    w_embed = jnp.concatenate(
        [conv_w[:, :, 0].T, conv_w[:, :, 1].T, conv_w[:, :, 2].T, temp_w.T],
        axis=0)
    pe = _positional_encoding(T, D)
    hdd = _embed(xcat, w_embed, pe)

    rot = jax.random.normal(jax.random.key(42), (DH, NH, NC // 2), jnp.float32)
    rot_flat = rot.reshape(DH, NH * (NC // 2))

    for l in range(NL):
        wq = qk_w[l].T.reshape(D, H, DH).transpose(1, 0, 2)
        wv = v_w[l].T.reshape(D, H, DH).transpose(1, 0, 2)
        wqkv = jnp.concatenate([wq, wv], axis=2)
        qkv, buckets = _qkv_hash(hdd, wqkv, rot_flat)
        sorted_qkv, sticker, pos = _sort_gather(buckets, qkv)
        so, sl = _attention(sorted_qkv)
        o, lg = _ungather(so, sl, pos)
        ao = _combine(o, lg)
        x1 = _outproj_ln(ao, out_w[l].T, out_b[l][None], hdd,
                         n1_s[l][None], n1_b[l][None])
        y = _ffn1(x1, ff1_w[l].T, ff1_b[l][None])
        hdd = _ffn2(y, ff2_w[l].T, ff2_b[l][None], x1,
                    n2_s[l][None], n2_b[l][None])

    out = _final(hdd, norm_s[None], norm_b[None], proj_w.T, proj_b[None], cout)
    return out[None, -PRED:, :]


# speedup vs baseline: 6.2235x; 5.0740x over previous
import functools
import numpy as np

import jax
import jax.numpy as jnp
from jax import lax
from jax.experimental import pallas as pl
from jax.experimental.pallas import tpu as pltpu
from jax.experimental.pallas import tpu_sc as plsc

B = 1
SEQ = 2048
PRED = 1024
ENC_IN = 7
D = 768
H = 12
DH = 64
DFF = 3072
NL = 2
BS = 64
NH = 4
T = SEQ + PRED
NC = T // BS
NPAIR = H * NH

_INTERPRET = False


def _pc(body, out_shape, in_specs, out_specs, grid=None, **kw):
    if grid is not None:
        kw["grid"] = grid
    return pl.pallas_call(
        body,
        out_shape=out_shape,
        in_specs=in_specs,
        out_specs=out_specs,
        interpret=_INTERPRET,
        **kw,
    )


def _positional_encoding(t, d):
    pos = np.arange(t, dtype=np.float32)[:, None]
    div = np.exp(np.arange(0, d, 2, dtype=np.float32) * -(np.log(10000.0) / d))
    pe = np.zeros((t, d), dtype=np.float32)
    pe[:, 0::2] = np.sin(pos * div)
    pe[:, 1::2] = np.cos(pos * div)
    return jnp.asarray(pe)


def _embed_body(xcat_ref, w_ref, pe_ref, out_ref):
    out_ref[...] = (
        jnp.dot(xcat_ref[...], w_ref[...], preferred_element_type=jnp.float32)
        + pe_ref[...]
    )


def _embed(xcat, w, pe):
    return _pc(
        _embed_body,
        jax.ShapeDtypeStruct((T, D), jnp.float32),
        [pl.BlockSpec(xcat.shape, lambda: (0, 0)),
         pl.BlockSpec(w.shape, lambda: (0, 0)),
         pl.BlockSpec(pe.shape, lambda: (0, 0))],
        pl.BlockSpec((T, D), lambda: (0, 0)),
    )(xcat, w, pe)


def _qkvh_body(hdd_ref, w_ref, rot_ref, qkv_ref, bkt_ref):
    qkv = jnp.dot(hdd_ref[...], w_ref[0], preferred_element_type=jnp.float32)
    qkv_ref[0] = qkv
    qk = qkv[:, :DH]
    s = jnp.dot(qk, rot_ref[...], preferred_element_type=jnp.float32)
    rows = []
    for n in range(NH):
        sn = s[:, n * (NC // 2):(n + 1) * (NC // 2)]
        cat = jnp.concatenate([sn, -sn], axis=1)
        bn = jnp.argmax(cat, axis=1).astype(jnp.int32)
        rows.append(bn.reshape(1, T))
    bkt_ref[0] = jnp.concatenate(rows, axis=0)


def _qkv_hash(hdd, wqkv, rot):
    return _pc(
        _qkvh_body,
        (jax.ShapeDtypeStruct((H, T, 2 * DH), jnp.float32),
         jax.ShapeDtypeStruct((H, NH, T), jnp.int32)),
        [pl.BlockSpec((T, D), lambda h: (0, 0)),
         pl.BlockSpec((1, D, 2 * DH), lambda h: (h, 0, 0)),
         pl.BlockSpec((DH, NH * (NC // 2)), lambda h: (0, 0))],
        (pl.BlockSpec((1, T, 2 * DH), lambda h: (h, 0, 0)),
         pl.BlockSpec((1, NH, T), lambda h: (h, 0, 0))),
        grid=(H,),
    )(hdd, wqkv, rot)


_SC_MESH = plsc.VectorSubcoreMesh(core_axis_name="c", subcore_axis_name="s")
_SC_PARAMS = pltpu.CompilerParams(needs_layout_passes=False)
_NW = 32
_VPB = T // 16
_GCH = 4
_GROWS = T // _GCH


def _sc_sort_body(bkt_hbm, qkv_hbm, pos_hbm, sqkv_hbm,
                  bktv, posv, stkv, histv, offv, gbuf, sem):
    wid = lax.axis_index("s") * 2 + lax.axis_index("c")

    def do_pair(pair):
        h = pair // NH
        pltpu.sync_copy(bkt_hbm.at[pl.ds(pair * T, T)], bktv)
        zero16 = jnp.zeros((16,), jnp.int32)
        for i in range(4):
            histv[pl.ds(i * 16, 16)] = zero16
        ones16 = zero16 + 1

        def p1(v, carry):
            b = bktv[pl.ds(v * 16, 16)]
            run = plsc.load_gather(histv, [b])
            u, _ = plsc.scan_count(b)
            posv[pl.ds(v * 16, 16)] = run + u - ones16
            ur, _ = plsc.scan_count(lax.rev(b, (0,)))
            is_last = lax.rev(ur, (0,)) == ones16
            plsc.addupdate_scatter(histv, [b], u, mask=is_last)
            return carry

        lax.fori_loop(0, _VPB, p1, 0)

        carry = jnp.int32(0)
        for i in range(3):
            hsl = histv[pl.ds(i * 16, 16)]
            inc = plsc.cumsum(hsl)
            offv[pl.ds(i * 16, 16)] = inc - hsl + carry
            carry = carry + jnp.sum(hsl)

        base_row = h * T

        def p2(v, carry):
            b = bktv[pl.ds(v * 16, 16)]
            p = posv[pl.ds(v * 16, 16)] + plsc.load_gather(offv, [b])
            posv[pl.ds(v * 16, 16)] = p
            rows = lax.iota(jnp.int32, 16) + (v * 16 + base_row)
            plsc.store_scatter(stkv, [p], rows)
            return carry

        lax.fori_loop(0, _VPB, p2, 0)
        pltpu.sync_copy(posv, pos_hbm.at[pl.ds(pair * T, T)])
        for c in range(_GCH):
            pltpu.async_copy(
                qkv_hbm.at[stkv.at[pl.ds(c * _GROWS, _GROWS)]], gbuf, sem
            ).wait()
            pltpu.sync_copy(
                gbuf, sqkv_hbm.at[pl.ds(pair * T + c * _GROWS, _GROWS)])

    do_pair(wid)

    @pl.when(wid < NPAIR - _NW)
    def _():
        do_pair(wid + _NW)


_sc_sort_gather = functools.partial(
    pl.kernel, _sc_sort_body, mesh=_SC_MESH, compiler_params=_SC_PARAMS,
    out_type=(jax.ShapeDtypeStruct((NPAIR * T,), jnp.int32),
              jax.ShapeDtypeStruct((NPAIR * T, 2 * DH), jnp.float32)),
    scratch_types=[pltpu.VMEM((T,), jnp.int32),
                   pltpu.VMEM((T,), jnp.int32),
                   pltpu.VMEM((T,), jnp.int32),
                   pltpu.VMEM((64,), jnp.int32),
                   pltpu.VMEM((64,), jnp.int32),
                   pltpu.VMEM((_GROWS, 2 * DH), jnp.float32),
                   pltpu.SemaphoreType.DMA],
)()


def _sc_ungather_body(so_hbm, pos_hbm, o_hbm, posv, absv, obuf, sem):
    wid = lax.axis_index("s") * 2 + lax.axis_index("c")

    def do_pair(pair):
        pltpu.sync_copy(pos_hbm.at[pl.ds(pair * T, T)], posv)
        base = pair * T

        def f(v, carry):
            absv[pl.ds(v * 16, 16)] = posv[pl.ds(v * 16, 16)] + base
            return carry

        lax.fori_loop(0, _VPB, f, 0)
        for c in range(_GCH):
            pltpu.async_copy(
                so_hbm.at[absv.at[pl.ds(c * _GROWS, _GROWS)]], obuf, sem
            ).wait()
            pltpu.sync_copy(
                obuf, o_hbm.at[pl.ds(pair * T + c * _GROWS, _GROWS)])

    do_pair(wid)

    @pl.when(wid < NPAIR - _NW)
    def _():
        do_pair(wid + _NW)


_sc_ungather = functools.partial(
    pl.kernel, _sc_ungather_body, mesh=_SC_MESH, compiler_params=_SC_PARAMS,
    out_type=jax.ShapeDtypeStruct((NPAIR * T, 2 * DH), jnp.float32),
    scratch_types=[pltpu.VMEM((T,), jnp.int32),
                   pltpu.VMEM((T,), jnp.int32),
                   pltpu.VMEM((_GROWS, 2 * DH), jnp.float32),
                   pltpu.SemaphoreType.DMA],
)()


def _sort_gather(buckets, qkv):
    pos_flat, sqkv_flat = _sc_sort_gather(
        buckets.reshape(NPAIR * T), qkv.reshape(H * T, 2 * DH))
    return sqkv_flat.reshape(NPAIR, T, 2 * DH), None, pos_flat


def _ungather(so, pos_flat):
    o_flat = _sc_ungather(so.reshape(NPAIR * T, 2 * DH), pos_flat)
    o = o_flat[:, :DH].reshape(NPAIR, T, DH)
    lg = o_flat[:, DH].reshape(NPAIR, 1, T)
    return o, lg


def _attn_body(qkv_ref, so_ref):
    qkv = qkv_ref[0]
    q = qkv[:, :DH]
    v = qkv[:, DH:]
    nrm = jnp.sqrt(jnp.sum(q * q, axis=1, keepdims=True)) + 1e-9
    k = q / nrm
    bq = q.reshape(NC, BS, DH)
    bk = k.reshape(NC, BS, DH)
    bv = v.reshape(NC, BS, DH)
    kp = jnp.concatenate([bk[NC - 1:], bk[:NC - 1]], axis=0)
    vp = jnp.concatenate([bv[NC - 1:], bv[:NC - 1]], axis=0)
    bk2 = jnp.concatenate([kp, bk], axis=1)
    bv2 = jnp.concatenate([vp, bv], axis=1)
    dots = lax.dot_general(
        bq, bk2, (((2,), (2,)), ((0,), (0,))),
        preferred_element_type=jnp.float32) / jnp.sqrt(jnp.float32(DH))
    ri = lax.broadcasted_iota(jnp.int32, (NC, BS, 2 * BS), 1)
    ci = lax.broadcasted_iota(jnp.int32, (NC, BS, 2 * BS), 2)
    dots = jnp.where(ci == ri + BS, -1e5, dots)
    m = jnp.max(dots, axis=-1, keepdims=True)
    lse = m + jnp.log(jnp.sum(jnp.exp(dots - m), axis=-1, keepdims=True))
    probs = jnp.exp(dots - lse)
    bo = lax.dot_general(
        probs, bv2, (((2,), (1,)), ((0,), (0,))),
        preferred_element_type=jnp.float32)
    lse_w = jnp.broadcast_to(lse, (NC, BS, DH)).reshape(T, DH)
    so_ref[0] = jnp.concatenate([bo.reshape(T, DH), lse_w], axis=1)


def _attention(sorted_qkv):
    return _pc(
        _attn_body,
        jax.ShapeDtypeStruct((NPAIR, T, 2 * DH), jnp.float32),
        [pl.BlockSpec((1, T, 2 * DH), lambda p: (p, 0, 0))],
        pl.BlockSpec((1, T, 2 * DH), lambda p: (p, 0, 0)),
        grid=(NPAIR,),
    )(sorted_qkv)


def _combine_body(o_ref, lg_ref, out_ref):
    cols = []
    for g in range(2):
        lg = lg_ref[g * NH:(g + 1) * NH, 0, :]
        m = jnp.max(lg, axis=0, keepdims=True)
        e = jnp.exp(lg - m)
        w = e / jnp.sum(e, axis=0, keepdims=True)
        o = o_ref[g * NH:(g + 1) * NH]
        cols.append(jnp.sum(o * w[:, :, None], axis=0))
    out_ref[...] = jnp.concatenate(cols, axis=1)


def _combine(o, lg):
    return _pc(
        _combine_body,
        jax.ShapeDtypeStruct((T, D), jnp.float32),
        [pl.BlockSpec((2 * NH, T, DH), lambda j: (j, 0, 0)),
         pl.BlockSpec((2 * NH, 1, T), lambda j: (j, 0, 0))],
        pl.BlockSpec((T, 2 * DH), lambda j: (0, j)),
        grid=(H // 2,),
    )(o, lg)


def _ln(x, s, b):
    m = jnp.mean(x, axis=-1, keepdims=True)
    v = jnp.mean((x - m) ** 2, axis=-1, keepdims=True)
    return (x - m) / jnp.sqrt(v + 1e-5) * s + b


def _outproj_body(ao_ref, w_ref, b_ref, hdd_ref, s_ref, bb_ref, out_ref):
    y = jnp.dot(ao_ref[...], w_ref[...], preferred_element_type=jnp.float32)
    y = y + b_ref[...] + hdd_ref[...]
    out_ref[...] = _ln(y, s_ref[...], bb_ref[...])


def _outproj_ln(ao, w, b, hdd, s, bb):
    return _pc(
        _outproj_body,
        jax.ShapeDtypeStruct((T, D), jnp.float32),
        [pl.BlockSpec((T, D), lambda: (0, 0)),
         pl.BlockSpec((D, D), lambda: (0, 0)),
         pl.BlockSpec((1, D), lambda: (0, 0)),
         pl.BlockSpec((T, D), lambda: (0, 0)),
         pl.BlockSpec((1, D), lambda: (0, 0)),
         pl.BlockSpec((1, D), lambda: (0, 0))],
        pl.BlockSpec((T, D), lambda: (0, 0)),
    )(ao, w, b, hdd, s, bb)


def _ffn1_body(x_ref, w_ref, b_ref, y_ref):
    h = jnp.dot(x_ref[...], w_ref[...], preferred_element_type=jnp.float32)
    h = h + b_ref[...]
    y_ref[...] = 0.5 * h * (1.0 + lax.erf(h / np.float32(np.sqrt(2.0))))


def _ffn1(x, w, b):
    nb = 4
    return _pc(
        _ffn1_body,
        jax.ShapeDtypeStruct((T, DFF), jnp.float32),
        [pl.BlockSpec((T, D), lambda j: (0, 0)),
         pl.BlockSpec((D, DFF // nb), lambda j: (0, j)),
         pl.BlockSpec((1, DFF // nb), lambda j: (0, j))],
        pl.BlockSpec((T, DFF // nb), lambda j: (0, j)),
        grid=(nb,),
    )(x, w, b)


def _ffn2_body(y_ref, w_ref, b_ref, x1_ref, s_ref, bb_ref, out_ref, *, nb):
    kidx = pl.program_id(0)
    part = jnp.dot(y_ref[...], w_ref[...], preferred_element_type=jnp.float32)

    @pl.when(kidx == 0)
    def _():
        out_ref[...] = part

    @pl.when(kidx > 0)
    def _():
        out_ref[...] += part

    @pl.when(kidx == nb - 1)
    def _():
        z = out_ref[...] + b_ref[...] + x1_ref[...]
        out_ref[...] = _ln(z, s_ref[...], bb_ref[...])


def _ffn2(y, w, b, x1, s, bb):
    nb = 8
    return _pc(
        functools.partial(_ffn2_body, nb=nb),
        jax.ShapeDtypeStruct((T, D), jnp.float32),
        [pl.BlockSpec((T, DFF // nb), lambda k: (0, k)),
         pl.BlockSpec((DFF // nb, D), lambda k: (k, 0)),
         pl.BlockSpec((1, D), lambda k: (0, 0)),
         pl.BlockSpec((T, D), lambda k: (0, 0)),
         pl.BlockSpec((1, D), lambda k: (0, 0)),
         pl.BlockSpec((1, D), lambda k: (0, 0))],
        pl.BlockSpec((T, D), lambda k: (0, 0)),
        grid=(nb,),
    )(y, w, b, x1, s, bb)


def _final_body(x_ref, s_ref, b_ref, w_ref, pb_ref, out_ref):
    x = _ln(x_ref[...], s_ref[...], b_ref[...])
    out_ref[...] = (jnp.dot(x, w_ref[...], preferred_element_type=jnp.float32)
                    + pb_ref[...])


def _final(x, s, b, w, pb, cout):
    return _pc(
        _final_body,
        jax.ShapeDtypeStruct((T, cout), jnp.float32),
        [pl.BlockSpec((T, D), lambda: (0, 0)),
         pl.BlockSpec((1, D), lambda: (0, 0)),
         pl.BlockSpec((1, D), lambda: (0, 0)),
         pl.BlockSpec((D, cout), lambda: (0, 0)),
         pl.BlockSpec((1, cout), lambda: (0, 0))],
        pl.BlockSpec((T, cout), lambda: (0, 0)),
    )(x, s, b, w, pb)


def kernel(x_enc, x_mark_enc, x_dec, x_mark_dec, conv_w, temp_w, qk_w, v_w,
           out_w, out_b, n1_s, n1_b, ff1_w, ff1_b, ff2_w, ff2_b, n2_s, n2_b,
           norm_s, norm_b, proj_w, proj_b):
    cout = proj_w.shape[0]
    x = jnp.concatenate([x_enc[0], x_dec[0, -PRED:, :]], axis=0)
    xm = jnp.concatenate([x_mark_enc[0], x_mark_dec[0, -PRED:, :]], axis=0)
    xprev = jnp.roll(x, 1, axis=0)
    xnext = jnp.roll(x, -1, axis=0)
    xcat = jnp.concatenate([xprev, x, xnext, xm], axis=1)
    w_embed = jnp.concatenate(
        [conv_w[:, :, 0].T, conv_w[:, :, 1].T, conv_w[:, :, 2].T, temp_w.T],
        axis=0)
    pe = _positional_encoding(T, D)
    hdd = _embed(xcat, w_embed, pe)

    rot = jax.random.normal(jax.random.key(42), (DH, NH, NC // 2), jnp.float32)
    rot_flat = rot.reshape(DH, NH * (NC // 2))

    for l in range(NL):
        wq = qk_w[l].T.reshape(D, H, DH).transpose(1, 0, 2)
        wv = v_w[l].T.reshape(D, H, DH).transpose(1, 0, 2)
        wqkv = jnp.concatenate([wq, wv], axis=2)
        qkv, buckets = _qkv_hash(hdd, wqkv, rot_flat)
        sorted_qkv, _, pos = _sort_gather(buckets, qkv)
        so = _attention(sorted_qkv)
        o, lg = _ungather(so, pos)
        ao = _combine(o, lg)
        x1 = _outproj_ln(ao, out_w[l].T, out_b[l][None], hdd,
                         n1_s[l][None], n1_b[l][None])
        y = _ffn1(x1, ff1_w[l].T, ff1_b[l][None])
        hdd = _ffn2(y, ff2_w[l].T, ff2_b[l][None], x1,
                    n2_s[l][None], n2_b[l][None])

    out = _final(hdd, norm_s[None], norm_b[None], proj_w.T, proj_b[None], cout)
    return out[None, -PRED:, :]


# speedup vs baseline: 6.3569x; 1.0214x over previous
import functools
import numpy as np

import jax
import jax.numpy as jnp
from jax import lax
from jax.experimental import pallas as pl
from jax.experimental.pallas import tpu as pltpu
from jax.experimental.pallas import tpu_sc as plsc

B = 1
SEQ = 2048
PRED = 1024
ENC_IN = 7
D = 768
H = 12
DH = 64
DFF = 3072
NL = 2
BS = 64
NH = 4
T = SEQ + PRED
NC = T // BS
NPAIR = H * NH

_INTERPRET = False


def _pc(body, out_shape, in_specs, out_specs, grid=None, **kw):
    if grid is not None:
        kw["grid"] = grid
    return pl.pallas_call(
        body,
        out_shape=out_shape,
        in_specs=in_specs,
        out_specs=out_specs,
        interpret=_INTERPRET,
        **kw,
    )


def _positional_encoding(t, d):
    pos = np.arange(t, dtype=np.float32)[:, None]
    div = np.exp(np.arange(0, d, 2, dtype=np.float32) * -(np.log(10000.0) / d))
    pe = np.zeros((t, d), dtype=np.float32)
    pe[:, 0::2] = np.sin(pos * div)
    pe[:, 1::2] = np.cos(pos * div)
    return jnp.asarray(pe)


def _embed_body(xcat_ref, w_ref, pe_ref, out_ref):
    out_ref[...] = (
        jnp.dot(xcat_ref[...], w_ref[...], preferred_element_type=jnp.float32)
        + pe_ref[...]
    )


def _embed(xcat, w, pe):
    return _pc(
        _embed_body,
        jax.ShapeDtypeStruct((T, D), jnp.float32),
        [pl.BlockSpec(xcat.shape, lambda: (0, 0)),
         pl.BlockSpec(w.shape, lambda: (0, 0)),
         pl.BlockSpec(pe.shape, lambda: (0, 0))],
        pl.BlockSpec((T, D), lambda: (0, 0)),
    )(xcat, w, pe)


def _qkvh_body(hdd_ref, w_ref, rot_ref, qkv_ref, bkt_ref):
    qkv = jnp.dot(hdd_ref[...], w_ref[0], preferred_element_type=jnp.float32)
    qkv_ref[0] = qkv
    qk = qkv[:, :DH]
    s = jnp.dot(qk, rot_ref[...], preferred_element_type=jnp.float32)
    rows = []
    for n in range(NH):
        sn = s[:, n * (NC // 2):(n + 1) * (NC // 2)]
        cat = jnp.concatenate([sn, -sn], axis=1)
        bn = jnp.argmax(cat, axis=1).astype(jnp.int32)
        rows.append(bn.reshape(1, T))
    bkt_ref[0] = jnp.concatenate(rows, axis=0)


def _qkv_hash(hdd, wqkv, rot):
    return _pc(
        _qkvh_body,
        (jax.ShapeDtypeStruct((H, T, 2 * DH), jnp.float32),
         jax.ShapeDtypeStruct((H, NH, T), jnp.int32)),
        [pl.BlockSpec((T, D), lambda h: (0, 0)),
         pl.BlockSpec((1, D, 2 * DH), lambda h: (h, 0, 0)),
         pl.BlockSpec((DH, NH * (NC // 2)), lambda h: (0, 0))],
        (pl.BlockSpec((1, T, 2 * DH), lambda h: (h, 0, 0)),
         pl.BlockSpec((1, NH, T), lambda h: (h, 0, 0))),
        grid=(H,),
    )(hdd, wqkv, rot)


_SC_MESH = plsc.VectorSubcoreMesh(core_axis_name="c", subcore_axis_name="s")
_SC_PARAMS = pltpu.CompilerParams(needs_layout_passes=False)
_NW = 32
_VPB = T // 16
_GCH = 4
_GROWS = T // _GCH


def _sc_sort_body(bkt_hbm, qkv_hbm, pos_hbm, sqkv_hbm,
                  bktv, posv, stkv, histv, offv, gbuf, sem):
    wid = lax.axis_index("s") * 2 + lax.axis_index("c")

    def do_pair(pair):
        h = pair // NH
        pltpu.sync_copy(bkt_hbm.at[pl.ds(pair * T, T)], bktv)
        zero16 = jnp.zeros((16,), jnp.int32)
        for i in range(4):
            histv[pl.ds(i * 16, 16)] = zero16
        ones16 = zero16 + 1

        def p1(v, carry):
            b = bktv[pl.ds(v * 16, 16)]
            run = plsc.load_gather(histv, [b])
            u, _ = plsc.scan_count(b)
            posv[pl.ds(v * 16, 16)] = run + u - ones16
            ur, _ = plsc.scan_count(lax.rev(b, (0,)))
            is_last = lax.rev(ur, (0,)) == ones16
            plsc.addupdate_scatter(histv, [b], u, mask=is_last)
            return carry

        lax.fori_loop(0, _VPB, p1, 0)

        carry = jnp.int32(0)
        for i in range(3):
            hsl = histv[pl.ds(i * 16, 16)]
            inc = plsc.cumsum(hsl)
            offv[pl.ds(i * 16, 16)] = inc - hsl + carry
            carry = carry + jnp.sum(hsl)

        base_row = h * T

        def p2(v, carry):
            b = bktv[pl.ds(v * 16, 16)]
            p = posv[pl.ds(v * 16, 16)] + plsc.load_gather(offv, [b])
            posv[pl.ds(v * 16, 16)] = p
            rows = lax.iota(jnp.int32, 16) + (v * 16 + base_row)
            plsc.store_scatter(stkv, [p], rows)
            return carry

        lax.fori_loop(0, _VPB, p2, 0)
        pltpu.sync_copy(posv, pos_hbm.at[pl.ds(pair * T, T)])
        for c in range(_GCH):
            pltpu.async_copy(
                qkv_hbm.at[stkv.at[pl.ds(c * _GROWS, _GROWS)]], gbuf, sem
            ).wait()
            pltpu.sync_copy(
                gbuf, sqkv_hbm.at[pl.ds(pair * T + c * _GROWS, _GROWS)])

    do_pair(wid)

    @pl.when(wid < NPAIR - _NW)
    def _():
        do_pair(wid + _NW)


_sc_sort_gather = functools.partial(
    pl.kernel, _sc_sort_body, mesh=_SC_MESH, compiler_params=_SC_PARAMS,
    out_type=(jax.ShapeDtypeStruct((NPAIR * T,), jnp.int32),
              jax.ShapeDtypeStruct((NPAIR * T, 2 * DH), jnp.float32)),
    scratch_types=[pltpu.VMEM((T,), jnp.int32),
                   pltpu.VMEM((T,), jnp.int32),
                   pltpu.VMEM((T,), jnp.int32),
                   pltpu.VMEM((64,), jnp.int32),
                   pltpu.VMEM((64,), jnp.int32),
                   pltpu.VMEM((_GROWS, 2 * DH), jnp.float32),
                   pltpu.SemaphoreType.DMA],
)()


def _sc_ungather_body(so_hbm, pos_hbm, o_hbm, posv, absv, obuf, sem):
    wid = lax.axis_index("s") * 2 + lax.axis_index("c")

    def do_pair(pair):
        pltpu.sync_copy(pos_hbm.at[pl.ds(pair * T, T)], posv)
        base = pair * T

        def f(v, carry):
            absv[pl.ds(v * 16, 16)] = posv[pl.ds(v * 16, 16)] + base
            return carry

        lax.fori_loop(0, _VPB, f, 0)
        for c in range(_GCH):
            pltpu.async_copy(
                so_hbm.at[absv.at[pl.ds(c * _GROWS, _GROWS)]], obuf, sem
            ).wait()
            pltpu.sync_copy(
                obuf, o_hbm.at[pl.ds(pair * T + c * _GROWS, _GROWS)])

    do_pair(wid)

    @pl.when(wid < NPAIR - _NW)
    def _():
        do_pair(wid + _NW)


_sc_ungather = functools.partial(
    pl.kernel, _sc_ungather_body, mesh=_SC_MESH, compiler_params=_SC_PARAMS,
    out_type=jax.ShapeDtypeStruct((NPAIR * T, 2 * DH), jnp.float32),
    scratch_types=[pltpu.VMEM((T,), jnp.int32),
                   pltpu.VMEM((T,), jnp.int32),
                   pltpu.VMEM((_GROWS, 2 * DH), jnp.float32),
                   pltpu.SemaphoreType.DMA],
)()


def _sort_gather(buckets, qkv):
    pos_flat, sqkv_flat = _sc_sort_gather(
        buckets.reshape(NPAIR * T), qkv.reshape(H * T, 2 * DH))
    return sqkv_flat.reshape(NPAIR, T, 2 * DH), None, pos_flat


def _ungather(so, pos_flat):
    o_flat = _sc_ungather(so.reshape(NPAIR * T, 2 * DH), pos_flat)
    o = o_flat[:, :DH].reshape(NPAIR, T, DH)
    lg = o_flat[:, DH].reshape(NPAIR, 1, T)
    return o, lg


def _attn_body(qkv_ref, so_ref):
    qkv = qkv_ref[0]
    q = qkv[:, :DH]
    v = qkv[:, DH:]
    nrm = jnp.sqrt(jnp.sum(q * q, axis=1, keepdims=True)) + 1e-9
    k = q / nrm
    bq = q.reshape(NC, BS, DH)
    bk = k.reshape(NC, BS, DH)
    bv = v.reshape(NC, BS, DH)
    kp = jnp.concatenate([bk[NC - 1:], bk[:NC - 1]], axis=0)
    vp = jnp.concatenate([bv[NC - 1:], bv[:NC - 1]], axis=0)
    bk2 = jnp.concatenate([kp, bk], axis=1)
    bv2 = jnp.concatenate([vp, bv], axis=1)
    dots = lax.dot_general(
        bq, bk2, (((2,), (2,)), ((0,), (0,))),
        preferred_element_type=jnp.float32) / jnp.sqrt(jnp.float32(DH))
    ri = lax.broadcasted_iota(jnp.int32, (NC, BS, 2 * BS), 1)
    ci = lax.broadcasted_iota(jnp.int32, (NC, BS, 2 * BS), 2)
    dots = jnp.where(ci == ri + BS, -1e5, dots)
    m = jnp.max(dots, axis=-1, keepdims=True)
    lse = m + jnp.log(jnp.sum(jnp.exp(dots - m), axis=-1, keepdims=True))
    probs = jnp.exp(dots - lse)
    bo = lax.dot_general(
        probs, bv2, (((2,), (1,)), ((0,), (0,))),
        preferred_element_type=jnp.float32)
    lse_w = jnp.broadcast_to(lse, (NC, BS, DH)).reshape(T, DH)
    so_ref[0] = jnp.concatenate([bo.reshape(T, DH), lse_w], axis=1)


def _attention(sorted_qkv):
    return _pc(
        _attn_body,
        jax.ShapeDtypeStruct((NPAIR, T, 2 * DH), jnp.float32),
        [pl.BlockSpec((1, T, 2 * DH), lambda p: (p, 0, 0))],
        pl.BlockSpec((1, T, 2 * DH), lambda p: (p, 0, 0)),
        grid=(NPAIR,),
    )(sorted_qkv)


def _combine_body(o_ref, lg_ref, out_ref):
    cols = []
    for g in range(2):
        lg = lg_ref[g * NH:(g + 1) * NH, 0, :]
        m = jnp.max(lg, axis=0, keepdims=True)
        e = jnp.exp(lg - m)
        w = e / jnp.sum(e, axis=0, keepdims=True)
        o = o_ref[g * NH:(g + 1) * NH]
        cols.append(jnp.sum(o * w[:, :, None], axis=0))
    out_ref[...] = jnp.concatenate(cols, axis=1)


def _combine(o, lg):
    return _pc(
        _combine_body,
        jax.ShapeDtypeStruct((T, D), jnp.float32),
        [pl.BlockSpec((2 * NH, T, DH), lambda j: (j, 0, 0)),
         pl.BlockSpec((2 * NH, 1, T), lambda j: (j, 0, 0))],
        pl.BlockSpec((T, 2 * DH), lambda j: (0, j)),
        grid=(H // 2,),
    )(o, lg)


def _ln(x, s, b):
    m = jnp.mean(x, axis=-1, keepdims=True)
    v = jnp.mean((x - m) ** 2, axis=-1, keepdims=True)
    return (x - m) / jnp.sqrt(v + 1e-5) * s + b


def _outproj_body(ao_ref, w_ref, b_ref, hdd_ref, s_ref, bb_ref, out_ref):
    y = jnp.dot(ao_ref[...].astype(jnp.bfloat16), w_ref[...],
                preferred_element_type=jnp.float32)
    y = y + b_ref[...] + hdd_ref[...]
    out_ref[...] = _ln(y, s_ref[...], bb_ref[...])


def _outproj_ln(ao, w, b, hdd, s, bb):
    return _pc(
        _outproj_body,
        jax.ShapeDtypeStruct((T, D), jnp.float32),
        [pl.BlockSpec((T, D), lambda: (0, 0)),
         pl.BlockSpec((D, D), lambda: (0, 0)),
         pl.BlockSpec((1, D), lambda: (0, 0)),
         pl.BlockSpec((T, D), lambda: (0, 0)),
         pl.BlockSpec((1, D), lambda: (0, 0)),
         pl.BlockSpec((1, D), lambda: (0, 0))],
        pl.BlockSpec((T, D), lambda: (0, 0)),
    )(ao, w, b, hdd, s, bb)


def _ffn1_body(x_ref, w_ref, b_ref, y_ref):
    h = jnp.dot(x_ref[...].astype(jnp.bfloat16), w_ref[...],
                preferred_element_type=jnp.float32)
    h = h + b_ref[...]
    g = 0.5 * h * (1.0 + lax.erf(h / np.float32(np.sqrt(2.0))))
    y_ref[...] = g.astype(jnp.bfloat16)


def _ffn1(x, w, b):
    nb = 4
    return _pc(
        _ffn1_body,
        jax.ShapeDtypeStruct((T, DFF), jnp.bfloat16),
        [pl.BlockSpec((T, D), lambda j: (0, 0)),
         pl.BlockSpec((D, DFF // nb), lambda j: (0, j)),
         pl.BlockSpec((1, DFF // nb), lambda j: (0, j))],
        pl.BlockSpec((T, DFF // nb), lambda j: (0, j)),
        grid=(nb,),
    )(x, w, b)


def _ffn2_body(y_ref, w_ref, b_ref, x1_ref, s_ref, bb_ref, out_ref, *, nb):
    kidx = pl.program_id(0)
    part = jnp.dot(y_ref[...], w_ref[...], preferred_element_type=jnp.float32)


    @pl.when(kidx == 0)
    def _():
        out_ref[...] = part

    @pl.when(kidx > 0)
    def _():
        out_ref[...] += part

    @pl.when(kidx == nb - 1)
    def _():
        z = out_ref[...] + b_ref[...] + x1_ref[...]
        out_ref[...] = _ln(z, s_ref[...], bb_ref[...])


def _ffn2(y, w, b, x1, s, bb):
    nb = 8
    return _pc(
        functools.partial(_ffn2_body, nb=nb),
        jax.ShapeDtypeStruct((T, D), jnp.float32),
        [pl.BlockSpec((T, DFF // nb), lambda k: (0, k)),
         pl.BlockSpec((DFF // nb, D), lambda k: (k, 0)),
         pl.BlockSpec((1, D), lambda k: (0, 0)),
         pl.BlockSpec((T, D), lambda k: (0, 0)),
         pl.BlockSpec((1, D), lambda k: (0, 0)),
         pl.BlockSpec((1, D), lambda k: (0, 0))],
        pl.BlockSpec((T, D), lambda k: (0, 0)),
        grid=(nb,),
    )(y, w, b, x1, s, bb)


def _final_body(x_ref, s_ref, b_ref, w_ref, pb_ref, out_ref):
    x = _ln(x_ref[...], s_ref[...], b_ref[...])
    out_ref[...] = (jnp.dot(x, w_ref[...], preferred_element_type=jnp.float32)
                    + pb_ref[...])


def _final(x, s, b, w, pb, cout):
    return _pc(
        _final_body,
        jax.ShapeDtypeStruct((T, cout), jnp.float32),
        [pl.BlockSpec((T, D), lambda: (0, 0)),
         pl.BlockSpec((1, D), lambda: (0, 0)),
         pl.BlockSpec((1, D), lambda: (0, 0)),
         pl.BlockSpec((D, cout), lambda: (0, 0)),
         pl.BlockSpec((1, cout), lambda: (0, 0))],
        pl.BlockSpec((T, cout), lambda: (0, 0)),
    )(x, s, b, w, pb)


def kernel(x_enc, x_mark_enc, x_dec, x_mark_dec, conv_w, temp_w, qk_w, v_w,
           out_w, out_b, n1_s, n1_b, ff1_w, ff1_b, ff2_w, ff2_b, n2_s, n2_b,
           norm_s, norm_b, proj_w, proj_b):
    cout = proj_w.shape[0]
    x = jnp.concatenate([x_enc[0], x_dec[0, -PRED:, :]], axis=0)
    xm = jnp.concatenate([x_mark_enc[0], x_mark_dec[0, -PRED:, :]], axis=0)
    xprev = jnp.roll(x, 1, axis=0)
    xnext = jnp.roll(x, -1, axis=0)
    xcat = jnp.concatenate([xprev, x, xnext, xm], axis=1)
    w_embed = jnp.concatenate(
        [conv_w[:, :, 0].T, conv_w[:, :, 1].T, conv_w[:, :, 2].T, temp_w.T],
        axis=0)
    pe = _positional_encoding(T, D)
    hdd = _embed(xcat, w_embed, pe)

    rot = jax.random.normal(jax.random.key(42), (DH, NH, NC // 2), jnp.float32)
    rot_flat = rot.reshape(DH, NH * (NC // 2))

    for l in range(NL):
        wq = qk_w[l].T.reshape(D, H, DH).transpose(1, 0, 2)
        wv = v_w[l].T.reshape(D, H, DH).transpose(1, 0, 2)
        wqkv = jnp.concatenate([wq, wv], axis=2)
        qkv, buckets = _qkv_hash(hdd, wqkv, rot_flat)
        sorted_qkv, _, pos = _sort_gather(buckets, qkv)
        so = _attention(sorted_qkv)
        o, lg = _ungather(so, pos)
        ao = _combine(o, lg)
        x1 = _outproj_ln(ao, out_w[l].T.astype(jnp.bfloat16), out_b[l][None],
                         hdd, n1_s[l][None], n1_b[l][None])
        y = _ffn1(x1, ff1_w[l].T.astype(jnp.bfloat16), ff1_b[l][None])
        hdd = _ffn2(y, ff2_w[l].T.astype(jnp.bfloat16), ff2_b[l][None], x1,
                    n2_s[l][None], n2_b[l][None])

    out = _final(hdd, norm_s[None], norm_b[None], proj_w.T, proj_b[None], cout)
    return out[None, -PRED:, :]


# speedup vs baseline: 7.7284x; 1.2157x over previous
import functools
import numpy as np

import jax
import jax.numpy as jnp
from jax import lax
from jax.experimental import pallas as pl
from jax.experimental.pallas import tpu as pltpu
from jax.experimental.pallas import tpu_sc as plsc

B = 1
SEQ = 2048
PRED = 1024
ENC_IN = 7
D = 768
H = 12
DH = 64
DFF = 3072
NL = 2
BS = 64
NH = 4
T = SEQ + PRED
NC = T // BS
NPAIR = H * NH

_INTERPRET = False


def _pc(body, out_shape, in_specs, out_specs, grid=None, **kw):
    if grid is not None:
        kw["grid"] = grid
    return pl.pallas_call(
        body,
        out_shape=out_shape,
        in_specs=in_specs,
        out_specs=out_specs,
        interpret=_INTERPRET,
        **kw,
    )


def _positional_encoding(t, d):
    pos = np.arange(t, dtype=np.float32)[:, None]
    div = np.exp(np.arange(0, d, 2, dtype=np.float32) * -(np.log(10000.0) / d))
    pe = np.zeros((t, d), dtype=np.float32)
    pe[:, 0::2] = np.sin(pos * div)
    pe[:, 1::2] = np.cos(pos * div)
    return jnp.asarray(pe)


def _embed_body(xcat_ref, w_ref, pe_ref, out_ref):
    out_ref[...] = (
        jnp.dot(xcat_ref[...], w_ref[...], preferred_element_type=jnp.float32)
        + pe_ref[...]
    )


def _embed(xcat, w, pe):
    return _pc(
        _embed_body,
        jax.ShapeDtypeStruct((T, D), jnp.float32),
        [pl.BlockSpec(xcat.shape, lambda: (0, 0)),
         pl.BlockSpec(w.shape, lambda: (0, 0)),
         pl.BlockSpec(pe.shape, lambda: (0, 0))],
        pl.BlockSpec((T, D), lambda: (0, 0)),
    )(xcat, w, pe)


def _qkvh_body(hdd_ref, w_ref, rot_ref, qkv_ref, bkt_ref):
    qkv = jnp.dot(hdd_ref[...], w_ref[0], preferred_element_type=jnp.float32)
    qkv_ref[0] = qkv
    qk = qkv[:, :DH]
    s = jnp.dot(qk, rot_ref[...], preferred_element_type=jnp.float32)
    pad = lax.broadcasted_iota(jnp.int32, (T, 128), 1) >= NC
    rows = []
    for n in range(NH):
        sn = jnp.where(pad, -1e30, s[:, n * 128:(n + 1) * 128])
        bn = jnp.argmax(sn, axis=1).astype(jnp.int32)
        rows.append(bn.reshape(1, T))
    bkt_ref[0] = jnp.concatenate(rows, axis=0)


def _qkv_hash(hdd, wqkv, rot):
    return _pc(
        _qkvh_body,
        (jax.ShapeDtypeStruct((H, T, 2 * DH), jnp.float32),
         jax.ShapeDtypeStruct((H, NH, T), jnp.int32)),
        [pl.BlockSpec((T, D), lambda h: (0, 0)),
         pl.BlockSpec((1, D, 2 * DH), lambda h: (h, 0, 0)),
         pl.BlockSpec((DH, NH * 128), lambda h: (0, 0))],
        (pl.BlockSpec((1, T, 2 * DH), lambda h: (h, 0, 0)),
         pl.BlockSpec((1, NH, T), lambda h: (h, 0, 0))),
        grid=(H,),
    )(hdd, wqkv, rot)


_SC_MESH = plsc.VectorSubcoreMesh(core_axis_name="c", subcore_axis_name="s")
_SC_PARAMS = pltpu.CompilerParams(needs_layout_passes=False)
_NW = 32
_VPB = T // 16
_GCH = 4
_GROWS = T // _GCH


def _sc_sort_body(bkt_hbm, qkv_hbm, pos_hbm, sqkv_hbm,
                  bktv, posv, stkv, histv, offv, gbuf, sem):
    wid = lax.axis_index("s") * 2 + lax.axis_index("c")

    def do_pair(pair):
        h = pair // NH
        pltpu.sync_copy(bkt_hbm.at[pl.ds(pair * T, T)], bktv)
        zero16 = jnp.zeros((16,), jnp.int32)
        for i in range(4):
            histv[pl.ds(i * 16, 16)] = zero16
        ones16 = zero16 + 1

        def p1(v, carry):
            b = bktv[pl.ds(v * 16, 16)]
            run = plsc.load_gather(histv, [b])
            u, _ = plsc.scan_count(b)
            posv[pl.ds(v * 16, 16)] = run + u - ones16
            ur, _ = plsc.scan_count(lax.rev(b, (0,)))
            is_last = lax.rev(ur, (0,)) == ones16
            plsc.addupdate_scatter(histv, [b], u, mask=is_last)
            return carry

        lax.fori_loop(0, _VPB, p1, 0)

        carry = jnp.int32(0)
        for i in range(3):
            hsl = histv[pl.ds(i * 16, 16)]
            inc = plsc.cumsum(hsl)
            offv[pl.ds(i * 16, 16)] = inc - hsl + carry
            carry = carry + jnp.sum(hsl)

        base_row = h * T

        def p2(v, carry):
            b = bktv[pl.ds(v * 16, 16)]
            p = posv[pl.ds(v * 16, 16)] + plsc.load_gather(offv, [b])
            posv[pl.ds(v * 16, 16)] = p
            rows = lax.iota(jnp.int32, 16) + (v * 16 + base_row)
            plsc.store_scatter(stkv, [p], rows)
            return carry

        lax.fori_loop(0, _VPB, p2, 0)
        pltpu.sync_copy(posv, pos_hbm.at[pl.ds(pair * T, T)])
        for c in range(_GCH):
            pltpu.async_copy(
                qkv_hbm.at[stkv.at[pl.ds(c * _GROWS, _GROWS)]], gbuf, sem
            ).wait()
            pltpu.sync_copy(
                gbuf, sqkv_hbm.at[pl.ds(pair * T + c * _GROWS, _GROWS)])

    do_pair(wid)

    @pl.when(wid < NPAIR - _NW)
    def _():
        do_pair(wid + _NW)


_sc_sort_gather = functools.partial(
    pl.kernel, _sc_sort_body, mesh=_SC_MESH, compiler_params=_SC_PARAMS,
    out_type=(jax.ShapeDtypeStruct((NPAIR * T,), jnp.int32),
              jax.ShapeDtypeStruct((NPAIR * T, 2 * DH), jnp.float32)),
    scratch_types=[pltpu.VMEM((T,), jnp.int32),
                   pltpu.VMEM((T,), jnp.int32),
                   pltpu.VMEM((T,), jnp.int32),
                   pltpu.VMEM((64,), jnp.int32),
                   pltpu.VMEM((64,), jnp.int32),
                   pltpu.VMEM((_GROWS, 2 * DH), jnp.float32),
                   pltpu.SemaphoreType.DMA],
)()


def _sc_ungather_body(so_hbm, pos_hbm, o_hbm, posv, absv, obuf, sem):
    wid = lax.axis_index("s") * 2 + lax.axis_index("c")

    def do_pair(pair):
        pltpu.sync_copy(pos_hbm.at[pl.ds(pair * T, T)], posv)
        base = pair * T

        def f(v, carry):
            absv[pl.ds(v * 16, 16)] = posv[pl.ds(v * 16, 16)] + base
            return carry

        lax.fori_loop(0, _VPB, f, 0)
        for c in range(_GCH):
            pltpu.async_copy(
                so_hbm.at[absv.at[pl.ds(c * _GROWS, _GROWS)]], obuf, sem
            ).wait()
            pltpu.sync_copy(
                obuf, o_hbm.at[pl.ds(pair * T + c * _GROWS, _GROWS)])

    do_pair(wid)

    @pl.when(wid < NPAIR - _NW)
    def _():
        do_pair(wid + _NW)


_sc_ungather = functools.partial(
    pl.kernel, _sc_ungather_body, mesh=_SC_MESH, compiler_params=_SC_PARAMS,
    out_type=jax.ShapeDtypeStruct((NPAIR * T, 2 * DH), jnp.float32),
    scratch_types=[pltpu.VMEM((T,), jnp.int32),
                   pltpu.VMEM((T,), jnp.int32),
                   pltpu.VMEM((_GROWS, 2 * DH), jnp.float32),
                   pltpu.SemaphoreType.DMA],
)()


def _sort_gather(buckets, qkv):
    pos_flat, sqkv_flat = _sc_sort_gather(
        buckets.reshape(NPAIR * T), qkv.reshape(H * T, 2 * DH))
    return sqkv_flat.reshape(NPAIR, T, 2 * DH), None, pos_flat


def _ungather(so, pos_flat):
    o_flat = _sc_ungather(so.reshape(NPAIR * T, 2 * DH), pos_flat)
    o = o_flat[:, :DH].reshape(NPAIR, T, DH)
    lg = o_flat[:, DH].reshape(NPAIR, 1, T)
    return o, lg


def _attn_body(qkv_ref, so_ref):
    qkv = qkv_ref[0]
    q = qkv[:, :DH]
    v = qkv[:, DH:]
    nrm = jnp.sqrt(jnp.sum(q * q, axis=1, keepdims=True)) + 1e-9
    k = q / nrm
    bq = q.reshape(NC, BS, DH)
    bk = k.reshape(NC, BS, DH)
    bv = v.reshape(NC, BS, DH)
    kp = jnp.concatenate([bk[NC - 1:], bk[:NC - 1]], axis=0)
    vp = jnp.concatenate([bv[NC - 1:], bv[:NC - 1]], axis=0)
    bk2 = jnp.concatenate([kp, bk], axis=1)
    bv2 = jnp.concatenate([vp, bv], axis=1)
    dots = lax.dot_general(
        bq, bk2, (((2,), (2,)), ((0,), (0,))),
        preferred_element_type=jnp.float32) / jnp.sqrt(jnp.float32(DH))
    ri = lax.broadcasted_iota(jnp.int32, (NC, BS, 2 * BS), 1)
    ci = lax.broadcasted_iota(jnp.int32, (NC, BS, 2 * BS), 2)
    dots = jnp.where(ci == ri + BS, -1e5, dots)
    m = jnp.max(dots, axis=-1, keepdims=True)
    lse = m + jnp.log(jnp.sum(jnp.exp(dots - m), axis=-1, keepdims=True))
    probs = jnp.exp(dots - lse)
    bo = lax.dot_general(
        probs, bv2, (((2,), (1,)), ((0,), (0,))),
        preferred_element_type=jnp.float32)
    lse_w = jnp.broadcast_to(lse, (NC, BS, DH)).reshape(T, DH)
    so_ref[0] = jnp.concatenate([bo.reshape(T, DH), lse_w], axis=1)


def _attention(sorted_qkv):
    return _pc(
        _attn_body,
        jax.ShapeDtypeStruct((NPAIR, T, 2 * DH), jnp.float32),
        [pl.BlockSpec((1, T, 2 * DH), lambda p: (p, 0, 0))],
        pl.BlockSpec((1, T, 2 * DH), lambda p: (p, 0, 0)),
        grid=(NPAIR,),
    )(sorted_qkv)


def _combine_body(o_ref, lg_ref, out_ref):
    cols = []
    for g in range(2):
        lg = lg_ref[g * NH:(g + 1) * NH, 0, :]
        m = jnp.max(lg, axis=0, keepdims=True)
        e = jnp.exp(lg - m)
        w = e / jnp.sum(e, axis=0, keepdims=True)
        o = o_ref[g * NH:(g + 1) * NH]
        cols.append(jnp.sum(o * w[:, :, None], axis=0))
    out_ref[...] = jnp.concatenate(cols, axis=1)


def _combine(o, lg):
    return _pc(
        _combine_body,
        jax.ShapeDtypeStruct((T, D), jnp.float32),
        [pl.BlockSpec((2 * NH, T, DH), lambda j: (j, 0, 0)),
         pl.BlockSpec((2 * NH, 1, T), lambda j: (j, 0, 0))],
        pl.BlockSpec((T, 2 * DH), lambda j: (0, j)),
        grid=(H // 2,),
    )(o, lg)


def _ln(x, s, b):
    m = jnp.mean(x, axis=-1, keepdims=True)
    v = jnp.mean((x - m) ** 2, axis=-1, keepdims=True)
    return (x - m) / jnp.sqrt(v + 1e-5) * s + b


def _outproj_body(ao_ref, w_ref, b_ref, hdd_ref, s_ref, bb_ref, out_ref):
    y = jnp.dot(ao_ref[...].astype(jnp.bfloat16), w_ref[...],
                preferred_element_type=jnp.float32)
    y = y + b_ref[...] + hdd_ref[...]
    out_ref[...] = _ln(y, s_ref[...], bb_ref[...])


def _outproj_ln(ao, w, b, hdd, s, bb):
    return _pc(
        _outproj_body,
        jax.ShapeDtypeStruct((T, D), jnp.float32),
        [pl.BlockSpec((T, D), lambda: (0, 0)),
         pl.BlockSpec((D, D), lambda: (0, 0)),
         pl.BlockSpec((1, D), lambda: (0, 0)),
         pl.BlockSpec((T, D), lambda: (0, 0)),
         pl.BlockSpec((1, D), lambda: (0, 0)),
         pl.BlockSpec((1, D), lambda: (0, 0))],
        pl.BlockSpec((T, D), lambda: (0, 0)),
    )(ao, w, b, hdd, s, bb)


def _ffn1_body(x_ref, w_ref, b_ref, y_ref):
    h = jnp.dot(x_ref[...].astype(jnp.bfloat16), w_ref[...],
                preferred_element_type=jnp.float32)
    h = h + b_ref[...]
    g = 0.5 * h * (1.0 + lax.erf(h / np.float32(np.sqrt(2.0))))
    y_ref[...] = g.astype(jnp.bfloat16)


def _ffn1(x, w, b):
    nb = 4
    return _pc(
        _ffn1_body,
        jax.ShapeDtypeStruct((T, DFF), jnp.bfloat16),
        [pl.BlockSpec((T, D), lambda j: (0, 0)),
         pl.BlockSpec((D, DFF // nb), lambda j: (0, j)),
         pl.BlockSpec((1, DFF // nb), lambda j: (0, j))],
        pl.BlockSpec((T, DFF // nb), lambda j: (0, j)),
        grid=(nb,),
    )(x, w, b)


def _ffn2_body(y_ref, w_ref, b_ref, x1_ref, s_ref, bb_ref, out_ref, *, nb):
    kidx = pl.program_id(0)
    part = jnp.dot(y_ref[...], w_ref[...], preferred_element_type=jnp.float32)


    @pl.when(kidx == 0)
    def _():
        out_ref[...] = part

    @pl.when(kidx > 0)
    def _():
        out_ref[...] += part

    @pl.when(kidx == nb - 1)
    def _():
        z = out_ref[...] + b_ref[...] + x1_ref[...]
        out_ref[...] = _ln(z, s_ref[...], bb_ref[...])


def _ffn2(y, w, b, x1, s, bb):
    nb = 8
    return _pc(
        functools.partial(_ffn2_body, nb=nb),
        jax.ShapeDtypeStruct((T, D), jnp.float32),
        [pl.BlockSpec((T, DFF // nb), lambda k: (0, k)),
         pl.BlockSpec((DFF // nb, D), lambda k: (k, 0)),
         pl.BlockSpec((1, D), lambda k: (0, 0)),
         pl.BlockSpec((T, D), lambda k: (0, 0)),
         pl.BlockSpec((1, D), lambda k: (0, 0)),
         pl.BlockSpec((1, D), lambda k: (0, 0))],
        pl.BlockSpec((T, D), lambda k: (0, 0)),
        grid=(nb,),
    )(y, w, b, x1, s, bb)


def _final_body(x_ref, s_ref, b_ref, w_ref, pb_ref, out_ref):
    x = _ln(x_ref[...], s_ref[...], b_ref[...])
    out_ref[...] = (jnp.dot(x, w_ref[...], preferred_element_type=jnp.float32)
                    + pb_ref[...])


def _final(x, s, b, w, pb, cout):
    return _pc(
        _final_body,
        jax.ShapeDtypeStruct((T, cout), jnp.float32),
        [pl.BlockSpec((T, D), lambda: (0, 0)),
         pl.BlockSpec((1, D), lambda: (0, 0)),
         pl.BlockSpec((1, D), lambda: (0, 0)),
         pl.BlockSpec((D, cout), lambda: (0, 0)),
         pl.BlockSpec((1, cout), lambda: (0, 0))],
        pl.BlockSpec((T, cout), lambda: (0, 0)),
    )(x, s, b, w, pb)


def kernel(x_enc, x_mark_enc, x_dec, x_mark_dec, conv_w, temp_w, qk_w, v_w,
           out_w, out_b, n1_s, n1_b, ff1_w, ff1_b, ff2_w, ff2_b, n2_s, n2_b,
           norm_s, norm_b, proj_w, proj_b):
    cout = proj_w.shape[0]
    x = jnp.concatenate([x_enc[0], x_dec[0, -PRED:, :]], axis=0)
    xm = jnp.concatenate([x_mark_enc[0], x_mark_dec[0, -PRED:, :]], axis=0)
    xprev = jnp.roll(x, 1, axis=0)
    xnext = jnp.roll(x, -1, axis=0)
    xcat = jnp.concatenate([xprev, x, xnext, xm], axis=1)
    w_embed = jnp.concatenate(
        [conv_w[:, :, 0].T, conv_w[:, :, 1].T, conv_w[:, :, 2].T, temp_w.T],
        axis=0)
    pe = _positional_encoding(T, D)
    hdd = _embed(xcat, w_embed, pe)

    rot = jax.random.normal(jax.random.key(42), (DH, NH, NC // 2), jnp.float32)
    rot2 = jnp.concatenate([rot, -rot], axis=2)
    rot_flat = jnp.pad(rot2, ((0, 0), (0, 0), (0, 128 - NC))).reshape(
        DH, NH * 128)

    for l in range(NL):
        wq = qk_w[l].T.reshape(D, H, DH).transpose(1, 0, 2)
        wv = v_w[l].T.reshape(D, H, DH).transpose(1, 0, 2)
        wqkv = jnp.concatenate([wq, wv], axis=2)
        qkv, buckets = _qkv_hash(hdd, wqkv, rot_flat)
        sorted_qkv, _, pos = _sort_gather(buckets, qkv)
        so = _attention(sorted_qkv)
        o, lg = _ungather(so, pos)
        ao = _combine(o, lg)
        x1 = _outproj_ln(ao, out_w[l].T.astype(jnp.bfloat16), out_b[l][None],
                         hdd, n1_s[l][None], n1_b[l][None])
        y = _ffn1(x1, ff1_w[l].T.astype(jnp.bfloat16), ff1_b[l][None])
        hdd = _ffn2(y, ff2_w[l].T.astype(jnp.bfloat16), ff2_b[l][None], x1,
                    n2_s[l][None], n2_b[l][None])

    out = _final(hdd, norm_s[None], norm_b[None], proj_w.T, proj_b[None], cout)
    return out[None, -PRED:, :]
